# baseline (speedup 1.0000x reference)
import functools

import jax
import jax.numpy as jnp
from jax import lax
from jax.experimental import pallas as pl
from jax.experimental.pallas import tpu as pltpu
from jax.experimental.pallas import tpu_sc as plsc

_NC = 2
_NS = 16
_NW = _NC * _NS
_GB = 4
_NB = 4
_L = 3
_RP = 32
_DP = 128


def _embed_lookup(xperm, table):
    b = xperm.shape[0]
    d = table.shape[1]
    per_w = b // _NW
    groups = per_w // _GB
    mesh = plsc.VectorSubcoreMesh(core_axis_name="c", subcore_axis_name="s")

    @functools.partial(
        pl.kernel,
        mesh=mesh,
        compiler_params=pltpu.CompilerParams(use_tc_tiling_on_sc=False),
        out_type=jax.ShapeDtypeStruct((b, _RP, _DP), jnp.float32),
        scratch_types=[
            pltpu.VMEM((per_w, 2 * _RP), jnp.int32),
            pltpu.VMEM((_NB, _GB, 2, _RP, 64), jnp.float32),
            pltpu.SemaphoreType.DMA((_NB,)),
            pltpu.SemaphoreType.DMA((_NB,)),
        ],
    )
    def run(x_hbm, table_hbm, out_hbm, idx_v, bufs, gsem, osem):
        wid = lax.axis_index("s") * _NC + lax.axis_index("c")
        batch0 = wid * per_w
        pltpu.sync_copy(x_hbm.at[pl.ds(batch0, per_w)], idx_v)

        def g_desc(g, rb, i, half):
            return pltpu.make_async_copy(
                table_hbm.at[idx_v.at[g * _GB + i, pl.ds(half * _RP, _RP)]],
                bufs.at[rb, i, half],
                gsem.at[rb],
            )

        def o_desc(g, rb, half):
            base = pl.multiple_of(batch0 + g * _GB, _GB)
            return pltpu.make_async_copy(
                bufs.at[rb, pl.ds(0, _GB), half],
                out_hbm.at[pl.ds(base, _GB), pl.ds(0, _RP), pl.ds(half * d, d)],
                osem.at[rb],
            )

        for g in range(_L):
            for i in range(_GB):
                for half in (0, 1):
                    g_desc(g, g % _NB, i, half).start()

        def outer(o, carry):
            for p in range(_NB):
                j = o * _NB + p
                gf = j + _L
                bf = (p + _L) % _NB

                @pl.when(gf < groups)
                def _fire():
                    @pl.when(gf >= _NB)
                    def _reuse():
                        for half in (0, 1):
                            o_desc(gf - _NB, bf, half).wait()

                    for i in range(_GB):
                        for half in (0, 1):
                            g_desc(gf, bf, i, half).start()

                for i in range(_GB):
                    for half in (0, 1):
                        g_desc(j, p, i, half).wait()
                for half in (0, 1):
                    o_desc(j, p, half).start()
            return carry

        lax.fori_loop(0, groups // _NB, outer, 0)

        for rb in range(_NB):
            for half in (0, 1):
                o_desc(groups - _NB + rb, rb, half).wait()

    return run(xperm, table)


def kernel(x, table):
    b, h = x.shape
    d = table.shape[1]
    xi = x.astype(jnp.int32)
    ev, od = xi[:, 0::2], xi[:, 1::2]
    fill = (_RP - (h + 1) // 2)
    xperm = jnp.concatenate(
        [ev, ev[:, :fill], od, od[:, :fill]], axis=1
    )
    outp = _embed_lookup(xperm, table)
    return outp[:, : h // 2, :].reshape(b, h, d)


# speedup vs baseline: 1.0272x; 1.0272x over previous
import functools

import jax
import jax.numpy as jnp
from jax import lax
from jax.experimental import pallas as pl
from jax.experimental.pallas import tpu as pltpu
from jax.experimental.pallas import tpu_sc as plsc

_NC = 2
_NS = 16
_NW = _NC * _NS
_GB = 4
_NB = 4
_L = 3
_HP = 56
_DP = 128


def _embed_lookup(x, table):
    b, h = x.shape
    d = table.shape[1]
    per_w = b // _NW
    groups = per_w // _GB
    mesh = plsc.VectorSubcoreMesh(core_axis_name="c", subcore_axis_name="s")

    @functools.partial(
        pl.kernel,
        mesh=mesh,
        compiler_params=pltpu.CompilerParams(use_tc_tiling_on_sc=False),
        out_type=jax.ShapeDtypeStruct((b, _HP, _DP), jnp.float32),
        scratch_types=[
            pltpu.VMEM((per_w, h), jnp.int32),
            pltpu.VMEM((_NB, _GB, _HP, d), jnp.float32),
            pltpu.SemaphoreType.DMA((_NB,)),
            pltpu.SemaphoreType.DMA((_NB,)),
        ],
    )
    def run(x_hbm, table_hbm, out_hbm, idx_v, bufs, gsem, osem):
        wid = lax.axis_index("s") * _NC + lax.axis_index("c")
        batch0 = wid * per_w
        pltpu.sync_copy(x_hbm.at[pl.ds(batch0, per_w)], idx_v)

        def g_desc(g, rb, i):
            return pltpu.make_async_copy(
                table_hbm.at[idx_v.at[g * _GB + i]],
                bufs.at[rb, i, pl.ds(0, h), pl.ds(0, d)],
                gsem.at[rb],
            )

        def o_desc(g, rb):
            base = pl.multiple_of(batch0 + g * _GB, _GB)
            return pltpu.make_async_copy(
                bufs.at[rb],
                out_hbm.at[pl.ds(base, _GB), pl.ds(0, _HP), pl.ds(0, d)],
                osem.at[rb],
            )

        for g in range(_L):
            for i in range(_GB):
                g_desc(g, g % _NB, i).start()

        def outer(o, carry):
            for p in range(_NB):
                j = o * _NB + p
                gf = j + _L
                bf = (p + _L) % _NB

                @pl.when(gf < groups)
                def _fire():
                    @pl.when(gf >= _NB)
                    def _reuse():
                        o_desc(gf - _NB, bf).wait()

                    for i in range(_GB):
                        g_desc(gf, bf, i).start()

                for i in range(_GB):
                    g_desc(j, p, i).wait()
                o_desc(j, p).start()
            return carry

        lax.fori_loop(0, groups // _NB, outer, 0)

        for rb in range(_NB):
            o_desc(groups - _NB + rb, rb).wait()

    return run(x, table)


def kernel(x, table):
    h = x.shape[1]
    d = table.shape[1]
    outp = _embed_lookup(x.astype(jnp.int32), table)
    return outp[:, :h, :d]


# speedup vs baseline: 1.0313x; 1.0040x over previous
import functools

import jax
import jax.numpy as jnp
from jax import lax
from jax.experimental import pallas as pl
from jax.experimental.pallas import tpu as pltpu
from jax.experimental.pallas import tpu_sc as plsc

_NC = 2
_NS = 16
_NW = _NC * _NS
_GB = 8
_NB = 4
_L = 3
_HP = 56
_DP = 128


def _embed_lookup(x, table):
    b, h = x.shape
    d = table.shape[1]
    per_w = b // _NW
    groups = per_w // _GB
    mesh = plsc.VectorSubcoreMesh(core_axis_name="c", subcore_axis_name="s")

    @functools.partial(
        pl.kernel,
        mesh=mesh,
        compiler_params=pltpu.CompilerParams(use_tc_tiling_on_sc=False),
        out_type=jax.ShapeDtypeStruct((b, _HP, _DP), jnp.float32),
        scratch_types=[
            pltpu.VMEM((per_w, h), jnp.int32),
            pltpu.VMEM((_NB, _GB, _HP, d), jnp.float32),
            pltpu.SemaphoreType.DMA((_NB,)),
            pltpu.SemaphoreType.DMA((_NB,)),
        ],
    )
    def run(x_hbm, table_hbm, out_hbm, idx_v, bufs, gsem, osem):
        wid = lax.axis_index("s") * _NC + lax.axis_index("c")
        batch0 = wid * per_w
        pltpu.sync_copy(x_hbm.at[pl.ds(batch0, per_w)], idx_v)

        def g_desc(g, rb, i):
            return pltpu.make_async_copy(
                table_hbm.at[idx_v.at[g * _GB + i]],
                bufs.at[rb, i, pl.ds(0, h), pl.ds(0, d)],
                gsem.at[rb],
            )

        def o_desc(g, rb):
            base = pl.multiple_of(batch0 + g * _GB, _GB)
            return pltpu.make_async_copy(
                bufs.at[rb],
                out_hbm.at[pl.ds(base, _GB), pl.ds(0, _HP), pl.ds(0, d)],
                osem.at[rb],
            )

        for g in range(_L):
            for i in range(_GB):
                g_desc(g, g % _NB, i).start()

        def outer(o, carry):
            for p in range(_NB):
                j = o * _NB + p
                gf = j + _L
                bf = (p + _L) % _NB

                @pl.when(gf < groups)
                def _fire():
                    @pl.when(gf >= _NB)
                    def _reuse():
                        o_desc(gf - _NB, bf).wait()

                    for i in range(_GB):
                        g_desc(gf, bf, i).start()

                for i in range(_GB):
                    g_desc(j, p, i).wait()
                o_desc(j, p).start()
            return carry

        lax.fori_loop(0, groups // _NB, outer, 0)

        for rb in range(_NB):
            o_desc(groups - _NB + rb, rb).wait()

    return run(x, table)


def kernel(x, table):
    h = x.shape[1]
    d = table.shape[1]
    outp = _embed_lookup(x.astype(jnp.int32), table)
    return outp[:, :h, :d]
